# baseline (speedup 1.0000x reference)
import functools
import math

import numpy as np
import jax
import jax.numpy as jnp
from jax import lax
from jax.experimental import pallas as pl
from jax.experimental.pallas import tpu as pltpu
from jax.experimental.pallas import tpu_sc as plsc

_N_IN = 2049
_N_OUT = 512
_OUTPUT_START = 20.0
_OUTPUT_END = 20000.0
_INPUT_END = 24000.0
_LANES = 16


def _op_constants():
    min_log = math.log10(1.0 + _OUTPUT_START)
    max_log = math.log10(1.0 + _OUTPUT_END)
    lin_logs = np.linspace(min_log, max_log, _N_OUT, dtype=np.float64)
    freq_per_bin = float(_INPUT_END) / (_N_IN - 1)
    center = ((np.power(10.0, lin_logs) - 1.0) / freq_per_bin).astype(np.float32)
    n_linear = 0
    while n_linear < _N_OUT - 1 and (
        center[n_linear + 1] - center[n_linear] <= 1.0 or center[n_linear] < 1.0
    ):
        n_linear += 1
    start_idx = center[:n_linear].astype(np.int64)
    frac_linear = center[:n_linear] - start_idx.astype(np.float32)
    n_sum = n_linear
    while n_sum < _N_OUT - 2 and (
        center[n_sum + 1] - center[n_sum] <= 2.0 or center[n_sum] < 2.0
    ):
        n_sum += 1
    n_cubic = n_sum - n_linear
    frac_cubic = center[n_linear : n_linear + n_cubic].copy()
    n_tri = _N_OUT - n_sum
    frac_tri = center[n_sum : n_sum + n_tri].copy()
    dist_tri = (center[n_sum : n_sum + n_tri] - center[n_sum - 1 : n_sum - 1 + n_tri]).copy()
    weights = np.full((n_tri, _N_IN), -np.inf, dtype=np.float32)
    for i in range(n_tri):
        i_mid = int(math.ceil(frac_tri[i]))
        i_start = int(math.ceil(frac_tri[i] - dist_tri[i]))
        i_end = int(math.ceil(frac_tri[i] + dist_tri[i + 1])) if i < n_tri - 1 else i_mid
        weights[i, i_mid] = 0.0
        dist_left = float(i_mid - i_start)
        for i_bin in range(i_start + 1, i_mid):
            lw = 1.0 - (i_mid - i_bin) / dist_left
            weights[i, i_bin] = np.float32(10.0 * np.log10(np.float32(lw)))
        if i_end > i_mid:
            dist_right = float(i_end - i_mid)
            for i_bin in range(i_mid + 1, i_end):
                lw = np.float32(1.0 - (i_bin - i_mid) / dist_right)
                weights[i, i_bin] = np.float32(10.0 * np.log10(lw))
    return n_linear, n_cubic, n_tri, start_idx, frac_linear, frac_cubic, weights


def _build_tables():
    n_linear, n_cubic, n_tri, lin_i0, frac_lin, frac_cub, wtri = _op_constants()
    n_sum = n_linear + n_cubic
    n_sum_groups = (n_sum + _LANES - 1) // _LANES
    sum_idx = np.zeros((n_sum_groups * _LANES, 4), np.int32)
    sum_w = np.zeros((n_sum_groups * _LANES, 4), np.float32)
    sum_idx[:n_linear, 0] = lin_i0
    sum_w[:n_linear, 0] = 1.0 - frac_lin
    sum_idx[:n_linear, 1] = lin_i0 + 1
    sum_w[:n_linear, 1] = frac_lin
    ci = np.floor(frac_cub).astype(np.int64)
    t = (frac_cub - ci.astype(np.float32)).astype(np.float32)
    cw = np.stack(
        [
            0.5 * (-t + 2 * t**2 - t**3),
            0.5 * (2 - 5 * t**2 + 3 * t**3),
            0.5 * (t + 4 * t**2 - 3 * t**3),
            0.5 * (-(t**2) + t**3),
        ],
        axis=1,
    ).astype(np.float32)
    for k in range(4):
        sum_idx[n_linear:n_sum, k] = np.clip(ci - 1 + k, 0, _N_IN - 1)
        sum_w[n_linear:n_sum, k] = cw[:, k]
    sum_idx = sum_idx.reshape(n_sum_groups, _LANES, 4).transpose(0, 2, 1).reshape(-1)
    sum_w = sum_w.reshape(n_sum_groups, _LANES, 4).transpose(0, 2, 1).reshape(-1)

    fin = np.isfinite(wtri)
    first = fin.argmax(axis=1)
    width = (fin.sum(axis=1)).astype(np.int64)
    tri_group_start = (n_sum // _LANES) * _LANES
    n_tri_groups = (_N_OUT - tri_group_start) // _LANES
    tidx_chunks, tw_chunks, group_k = [], [], []
    for g in range(n_tri_groups):
        outs = np.arange(tri_group_start + g * _LANES, tri_group_start + (g + 1) * _LANES)
        tri_i = outs - n_sum
        real = tri_i >= 0
        k_g = int(width[tri_i[real]].max())
        gi = np.zeros((k_g, _LANES), np.int32)
        gw = np.full((k_g, _LANES), -np.inf, np.float32)
        for lane in range(_LANES):
            if not real[lane]:
                continue
            i = tri_i[lane]
            f0, wd = int(first[i]), int(width[i])
            gi[:wd, lane] = np.arange(f0, f0 + wd, dtype=np.int32)
            gw[:wd, lane] = wtri[i, f0 : f0 + wd]
        tidx_chunks.append(gi.reshape(-1))
        tw_chunks.append(gw.reshape(-1))
        group_k.append(k_g)
    tri_idx = np.concatenate(tidx_chunks)
    tri_w = np.concatenate(tw_chunks)
    return (
        n_sum_groups,
        n_sum - (n_sum // _LANES) * _LANES,
        tuple(group_k),
        sum_idx,
        sum_w,
        tri_idx,
        tri_w,
    )


(_N_SUM_GROUPS, _N_BOUNDARY, _TRI_K, _SUM_IDX, _SUM_W, _TRI_IDX, _TRI_W) = _build_tables()

_NUM_CORES = 2
_NUM_SUBCORES = 16
_NW = _NUM_CORES * _NUM_SUBCORES
_ROWS = 8 * 256
_RPW = _ROWS // _NW
_BLK = 16
_NBLK = _RPW // _BLK
_RB = 8


def _sc_body(x_hbm, sidx_hbm, sw_hbm, tidx_hbm, tw_hbm, out_hbm, xb, ob, sidx, sw, tidx, tw):
    cid = lax.axis_index("c")
    sid = lax.axis_index("s")
    wid = sid * _NUM_CORES + cid
    pltpu.sync_copy(sidx_hbm, sidx)
    pltpu.sync_copy(sw_hbm, sw)
    pltpu.sync_copy(tidx_hbm, tidx)
    pltpu.sync_copy(tw_hbm, tw)

    def do_subblock(rows):
        rbase = [jnp.full((_LANES,), r, jnp.int32) for r in rows]

        def sum_group(g, _):
            accs = [jnp.zeros((_LANES,), jnp.float32) for _ in rows]
            for k in range(4):
                off = (g * 4 + k) * _LANES
                iv = sidx[pl.ds(off, _LANES)]
                wv = sw[pl.ds(off, _LANES)]
                for j, rb in enumerate(rbase):
                    accs[j] = accs[j] + wv * plsc.load_gather(xb, [rb, iv])
            goff = g * _LANES
            for j, r in enumerate(rows):
                ob[r, pl.ds(goff, _LANES)] = accs[j]
            return _

        lax.fori_loop(0, _N_SUM_GROUPS - 1, sum_group, None)

        baccs = [jnp.zeros((_LANES,), jnp.float32) for _ in rows]
        gb = _N_SUM_GROUPS - 1
        for k in range(4):
            off = (gb * 4 + k) * _LANES
            iv = sidx[pl.ds(off, _LANES)]
            wv = sw[pl.ds(off, _LANES)]
            for j, rb in enumerate(rbase):
                baccs[j] = baccs[j] + wv * plsc.load_gather(xb, [rb, iv])

        toff = 0
        for tg, k_g in enumerate(_TRI_K):
            base = toff

            def tri_tap(k, accs, base=base):
                off = (base + k) * _LANES
                iv = tidx[pl.ds(off, _LANES)]
                wv = tw[pl.ds(off, _LANES)]
                return tuple(
                    jnp.maximum(a, wv + plsc.load_gather(xb, [rb, iv]))
                    for a, rb in zip(accs, rbase)
                )

            init = tuple(jnp.full((_LANES,), -jnp.inf, jnp.float32) for _ in rows)
            accs = lax.fori_loop(0, k_g, tri_tap, init)
            toff += k_g
            if tg == 0:
                lane = lax.iota(jnp.int32, _LANES)
                accs = tuple(
                    jnp.where(lane < _N_BOUNDARY, b, a) for a, b in zip(accs, baccs)
                )
            for j, r in enumerate(rows):
                ob[r, pl.ds((gb + tg) * _LANES, _LANES)] = accs[j]

    batch = wid // (256 // _RPW)
    frame0 = (wid % (256 // _RPW)) * _RPW

    def do_block(blk, _):
        fr = frame0 + blk * _BLK
        pltpu.sync_copy(x_hbm.at[batch, pl.ds(fr, _BLK)], xb)
        for sub in range(_BLK // _RB):
            do_subblock(list(range(sub * _RB, (sub + 1) * _RB)))
        pltpu.sync_copy(ob, out_hbm.at[batch, pl.ds(fr, _BLK)])
        return _

    lax.fori_loop(0, _NBLK, do_block, None)


@jax.jit
def _log_scale_sc(x3d):
    n_tri_taps = _TRI_IDX.shape[0] // _LANES
    call = pl.kernel(
        _sc_body,
        out_type=jax.ShapeDtypeStruct((8, 256, _N_OUT), jnp.float32),
        mesh=plsc.VectorSubcoreMesh(
            core_axis_name="c",
            subcore_axis_name="s",
            num_cores=_NUM_CORES,
            num_subcores=_NUM_SUBCORES,
        ),
        compiler_params=pltpu.CompilerParams(
            use_tc_tiling_on_sc=False,
            needs_layout_passes=False,
            disable_bounds_checks=True,
        ),
        scratch_types=[
            pltpu.VMEM((_BLK, _N_IN), jnp.float32),
            pltpu.VMEM((_BLK, _N_OUT), jnp.float32),
            pltpu.VMEM((_N_SUM_GROUPS * 4 * _LANES,), jnp.int32),
            pltpu.VMEM((_N_SUM_GROUPS * 4 * _LANES,), jnp.float32),
            pltpu.VMEM((n_tri_taps * _LANES,), jnp.int32),
            pltpu.VMEM((n_tri_taps * _LANES,), jnp.float32),
        ],
    )
    return call(x3d, _SUM_IDX, _SUM_W, _TRI_IDX, _TRI_W)


def kernel(x):
    return _log_scale_sc(x)


# speedup vs baseline: 1.4652x; 1.4652x over previous
import functools
import math

import numpy as np
import jax
import jax.numpy as jnp
from jax import lax
from jax.experimental import pallas as pl
from jax.experimental.pallas import tpu as pltpu
from jax.experimental.pallas import tpu_sc as plsc

_N_IN = 2049
_N_OUT = 512
_OUTPUT_START = 20.0
_OUTPUT_END = 20000.0
_INPUT_END = 24000.0
_LANES = 16


def _op_constants():
    min_log = math.log10(1.0 + _OUTPUT_START)
    max_log = math.log10(1.0 + _OUTPUT_END)
    lin_logs = np.linspace(min_log, max_log, _N_OUT, dtype=np.float64)
    freq_per_bin = float(_INPUT_END) / (_N_IN - 1)
    center = ((np.power(10.0, lin_logs) - 1.0) / freq_per_bin).astype(np.float32)
    n_linear = 0
    while n_linear < _N_OUT - 1 and (
        center[n_linear + 1] - center[n_linear] <= 1.0 or center[n_linear] < 1.0
    ):
        n_linear += 1
    start_idx = center[:n_linear].astype(np.int64)
    frac_linear = center[:n_linear] - start_idx.astype(np.float32)
    n_sum = n_linear
    while n_sum < _N_OUT - 2 and (
        center[n_sum + 1] - center[n_sum] <= 2.0 or center[n_sum] < 2.0
    ):
        n_sum += 1
    n_cubic = n_sum - n_linear
    frac_cubic = center[n_linear : n_linear + n_cubic].copy()
    n_tri = _N_OUT - n_sum
    frac_tri = center[n_sum : n_sum + n_tri].copy()
    dist_tri = (center[n_sum : n_sum + n_tri] - center[n_sum - 1 : n_sum - 1 + n_tri]).copy()
    weights = np.full((n_tri, _N_IN), -np.inf, dtype=np.float32)
    for i in range(n_tri):
        i_mid = int(math.ceil(frac_tri[i]))
        i_start = int(math.ceil(frac_tri[i] - dist_tri[i]))
        i_end = int(math.ceil(frac_tri[i] + dist_tri[i + 1])) if i < n_tri - 1 else i_mid
        weights[i, i_mid] = 0.0
        dist_left = float(i_mid - i_start)
        for i_bin in range(i_start + 1, i_mid):
            lw = 1.0 - (i_mid - i_bin) / dist_left
            weights[i, i_bin] = np.float32(10.0 * np.log10(np.float32(lw)))
        if i_end > i_mid:
            dist_right = float(i_end - i_mid)
            for i_bin in range(i_mid + 1, i_end):
                lw = np.float32(1.0 - (i_bin - i_mid) / dist_right)
                weights[i, i_bin] = np.float32(10.0 * np.log10(lw))
    return n_linear, n_cubic, n_tri, start_idx, frac_linear, frac_cubic, weights


def _build_tables():
    n_linear, n_cubic, n_tri, lin_i0, frac_lin, frac_cub, wtri = _op_constants()
    n_sum = n_linear + n_cubic
    n_sum_groups = (n_sum + _LANES - 1) // _LANES
    sum_idx = np.zeros((n_sum_groups * _LANES, 4), np.int32)
    sum_w = np.zeros((n_sum_groups * _LANES, 4), np.float32)
    sum_idx[:n_linear, 0] = lin_i0
    sum_w[:n_linear, 0] = 1.0 - frac_lin
    sum_idx[:n_linear, 1] = lin_i0 + 1
    sum_w[:n_linear, 1] = frac_lin
    ci = np.floor(frac_cub).astype(np.int64)
    t = (frac_cub - ci.astype(np.float32)).astype(np.float32)
    cw = np.stack(
        [
            0.5 * (-t + 2 * t**2 - t**3),
            0.5 * (2 - 5 * t**2 + 3 * t**3),
            0.5 * (t + 4 * t**2 - 3 * t**3),
            0.5 * (-(t**2) + t**3),
        ],
        axis=1,
    ).astype(np.float32)
    for k in range(4):
        sum_idx[n_linear:n_sum, k] = np.clip(ci - 1 + k, 0, _N_IN - 1)
        sum_w[n_linear:n_sum, k] = cw[:, k]
    sum_idx = sum_idx.reshape(n_sum_groups, _LANES, 4).transpose(0, 2, 1).reshape(-1)
    sum_w = sum_w.reshape(n_sum_groups, _LANES, 4).transpose(0, 2, 1).reshape(-1)

    fin = np.isfinite(wtri)
    first = fin.argmax(axis=1)
    width = (fin.sum(axis=1)).astype(np.int64)
    tri_group_start = (n_sum // _LANES) * _LANES
    n_tri_groups = (_N_OUT - tri_group_start) // _LANES
    tidx_chunks, tw_chunks, group_k = [], [], []
    for g in range(n_tri_groups):
        outs = np.arange(tri_group_start + g * _LANES, tri_group_start + (g + 1) * _LANES)
        tri_i = outs - n_sum
        real = tri_i >= 0
        k_g = int(width[tri_i[real]].max())
        gi = np.zeros((k_g, _LANES), np.int32)
        gw = np.full((k_g, _LANES), -np.inf, np.float32)
        for lane in range(_LANES):
            if not real[lane]:
                continue
            i = tri_i[lane]
            f0, wd = int(first[i]), int(width[i])
            gi[:wd, lane] = np.arange(f0, f0 + wd, dtype=np.int32)
            gw[:wd, lane] = wtri[i, f0 : f0 + wd]
        tidx_chunks.append(gi.reshape(-1))
        tw_chunks.append(gw.reshape(-1))
        group_k.append(k_g)
    tri_idx = np.concatenate(tidx_chunks)
    tri_w = np.concatenate(tw_chunks)
    return (
        n_sum_groups,
        n_sum - (n_sum // _LANES) * _LANES,
        tuple(group_k),
        sum_idx,
        sum_w,
        tri_idx,
        tri_w,
    )


(_N_SUM_GROUPS, _N_BOUNDARY, _TRI_K, _SUM_IDX, _SUM_W, _TRI_IDX, _TRI_W) = _build_tables()

_NUM_CORES = 2
_NUM_SUBCORES = 16
_NW = _NUM_CORES * _NUM_SUBCORES
_ROWS = 8 * 256
_RPW = _ROWS // _NW
_BLK = 16
_NBLK = _RPW // _BLK
_RB = 8


def _sc_body(x_hbm, sidx_hbm, sw_hbm, tidx_hbm, tw_hbm, out_hbm, xb, ob, sidx, sw, tidx, tw):
    cid = lax.axis_index("c")
    sid = lax.axis_index("s")
    wid = sid * _NUM_CORES + cid
    pltpu.sync_copy(sidx_hbm, sidx)
    pltpu.sync_copy(sw_hbm, sw)
    pltpu.sync_copy(tidx_hbm, tidx)
    pltpu.sync_copy(tw_hbm, tw)

    def do_subblock(rows):
        rbase = [jnp.full((_LANES,), r, jnp.int32) for r in rows]

        def sum_group(g, _):
            accs = [jnp.zeros((_LANES,), jnp.float32) for _ in rows]
            for k in range(4):
                off = (g * 4 + k) * _LANES
                iv = sidx[pl.ds(off, _LANES)]
                wv = sw[pl.ds(off, _LANES)]
                for j, rb in enumerate(rbase):
                    accs[j] = accs[j] + wv * plsc.load_gather(xb, [rb, iv])
            goff = g * _LANES
            for j, r in enumerate(rows):
                ob[r, pl.ds(goff, _LANES)] = accs[j]
            return _

        lax.fori_loop(0, _N_SUM_GROUPS - 1, sum_group, None)

        baccs = [jnp.zeros((_LANES,), jnp.float32) for _ in rows]
        gb = _N_SUM_GROUPS - 1
        for k in range(4):
            off = (gb * 4 + k) * _LANES
            iv = sidx[pl.ds(off, _LANES)]
            wv = sw[pl.ds(off, _LANES)]
            for j, rb in enumerate(rbase):
                baccs[j] = baccs[j] + wv * plsc.load_gather(xb, [rb, iv])

        toff = 0
        for tg, k_g in enumerate(_TRI_K):
            base = toff

            def tri_tap(k, accs, base=base):
                off = (base + k) * _LANES
                iv = tidx[pl.ds(off, _LANES)]
                wv = tw[pl.ds(off, _LANES)]
                return tuple(
                    jnp.maximum(a, wv + plsc.load_gather(xb, [rb, iv]))
                    for a, rb in zip(accs, rbase)
                )

            init = tuple(jnp.full((_LANES,), -jnp.inf, jnp.float32) for _ in rows)
            accs = lax.fori_loop(0, k_g, tri_tap, init)
            toff += k_g
            if tg == 0:
                lane = lax.iota(jnp.int32, _LANES)
                accs = tuple(
                    jnp.where(lane < _N_BOUNDARY, b, a) for a, b in zip(accs, baccs)
                )
            for j, r in enumerate(rows):
                ob[r, pl.ds((gb + tg) * _LANES, _LANES)] = accs[j]

    batch = wid // (256 // _RPW)
    frame0 = (wid % (256 // _RPW)) * _RPW

    def do_block(blk, _):
        fr = frame0 + blk * _BLK
        pltpu.sync_copy(x_hbm.at[batch, pl.ds(fr, _BLK)], xb)
        for sub in range(_BLK // _RB):
            do_subblock(list(range(sub * _RB, (sub + 1) * _RB)))
        pltpu.sync_copy(ob, out_hbm.at[batch, pl.ds(fr, _BLK)])
        return _

    lax.fori_loop(0, _NBLK, do_block, None)


@jax.jit
def _log_scale_sc(x3d):
    n_tri_taps = _TRI_IDX.shape[0] // _LANES
    call = pl.kernel(
        _sc_body,
        out_type=jax.ShapeDtypeStruct((8, 256, _N_OUT), jnp.float32),
        mesh=plsc.VectorSubcoreMesh(
            core_axis_name="c",
            subcore_axis_name="s",
            num_cores=_NUM_CORES,
            num_subcores=_NUM_SUBCORES,
        ),
        compiler_params=pltpu.CompilerParams(
            use_tc_tiling_on_sc=True,
            needs_layout_passes=False,
            disable_bounds_checks=True,
        ),
        scratch_types=[
            pltpu.VMEM((_BLK, _N_IN), jnp.float32),
            pltpu.VMEM((_BLK, _N_OUT), jnp.float32),
            pltpu.VMEM((_N_SUM_GROUPS * 4 * _LANES,), jnp.int32),
            pltpu.VMEM((_N_SUM_GROUPS * 4 * _LANES,), jnp.float32),
            pltpu.VMEM((n_tri_taps * _LANES,), jnp.int32),
            pltpu.VMEM((n_tri_taps * _LANES,), jnp.float32),
        ],
    )
    return call(x3d, _SUM_IDX, _SUM_W, _TRI_IDX, _TRI_W)


def kernel(x):
    return _log_scale_sc(x)


# speedup vs baseline: 1.5299x; 1.0442x over previous
import functools
import math

import numpy as np
import jax
import jax.numpy as jnp
from jax import lax
from jax.experimental import pallas as pl
from jax.experimental.pallas import tpu as pltpu
from jax.experimental.pallas import tpu_sc as plsc

_N_IN = 2049
_N_OUT = 512
_OUTPUT_START = 20.0
_OUTPUT_END = 20000.0
_INPUT_END = 24000.0
_LANES = 16


def _op_constants():
    min_log = math.log10(1.0 + _OUTPUT_START)
    max_log = math.log10(1.0 + _OUTPUT_END)
    lin_logs = np.linspace(min_log, max_log, _N_OUT, dtype=np.float64)
    freq_per_bin = float(_INPUT_END) / (_N_IN - 1)
    center = ((np.power(10.0, lin_logs) - 1.0) / freq_per_bin).astype(np.float32)
    n_linear = 0
    while n_linear < _N_OUT - 1 and (
        center[n_linear + 1] - center[n_linear] <= 1.0 or center[n_linear] < 1.0
    ):
        n_linear += 1
    start_idx = center[:n_linear].astype(np.int64)
    frac_linear = center[:n_linear] - start_idx.astype(np.float32)
    n_sum = n_linear
    while n_sum < _N_OUT - 2 and (
        center[n_sum + 1] - center[n_sum] <= 2.0 or center[n_sum] < 2.0
    ):
        n_sum += 1
    n_cubic = n_sum - n_linear
    frac_cubic = center[n_linear : n_linear + n_cubic].copy()
    n_tri = _N_OUT - n_sum
    frac_tri = center[n_sum : n_sum + n_tri].copy()
    dist_tri = (center[n_sum : n_sum + n_tri] - center[n_sum - 1 : n_sum - 1 + n_tri]).copy()
    weights = np.full((n_tri, _N_IN), -np.inf, dtype=np.float32)
    for i in range(n_tri):
        i_mid = int(math.ceil(frac_tri[i]))
        i_start = int(math.ceil(frac_tri[i] - dist_tri[i]))
        i_end = int(math.ceil(frac_tri[i] + dist_tri[i + 1])) if i < n_tri - 1 else i_mid
        weights[i, i_mid] = 0.0
        dist_left = float(i_mid - i_start)
        for i_bin in range(i_start + 1, i_mid):
            lw = 1.0 - (i_mid - i_bin) / dist_left
            weights[i, i_bin] = np.float32(10.0 * np.log10(np.float32(lw)))
        if i_end > i_mid:
            dist_right = float(i_end - i_mid)
            for i_bin in range(i_mid + 1, i_end):
                lw = np.float32(1.0 - (i_bin - i_mid) / dist_right)
                weights[i, i_bin] = np.float32(10.0 * np.log10(lw))
    return n_linear, n_cubic, n_tri, start_idx, frac_linear, frac_cubic, weights


def _build_tables():
    n_linear, n_cubic, n_tri, lin_i0, frac_lin, frac_cub, wtri = _op_constants()
    n_sum = n_linear + n_cubic
    n_sum_groups = (n_sum + _LANES - 1) // _LANES
    sum_idx = np.zeros((n_sum_groups * _LANES, 4), np.int32)
    sum_w = np.zeros((n_sum_groups * _LANES, 4), np.float32)
    sum_idx[:n_linear, 0] = lin_i0
    sum_w[:n_linear, 0] = 1.0 - frac_lin
    sum_idx[:n_linear, 1] = lin_i0 + 1
    sum_w[:n_linear, 1] = frac_lin
    ci = np.floor(frac_cub).astype(np.int64)
    t = (frac_cub - ci.astype(np.float32)).astype(np.float32)
    cw = np.stack(
        [
            0.5 * (-t + 2 * t**2 - t**3),
            0.5 * (2 - 5 * t**2 + 3 * t**3),
            0.5 * (t + 4 * t**2 - 3 * t**3),
            0.5 * (-(t**2) + t**3),
        ],
        axis=1,
    ).astype(np.float32)
    for k in range(4):
        sum_idx[n_linear:n_sum, k] = np.clip(ci - 1 + k, 0, _N_IN - 1)
        sum_w[n_linear:n_sum, k] = cw[:, k]
    n_lin2_groups = n_linear // _LANES
    sum_idx4 = sum_idx.reshape(n_sum_groups, _LANES, 4).transpose(0, 2, 1)
    sum_w4 = sum_w.reshape(n_sum_groups, _LANES, 4).transpose(0, 2, 1)
    lin2_idx = sum_idx4[:n_lin2_groups, :2].reshape(-1)
    lin2_w = sum_w4[:n_lin2_groups, :2].reshape(-1)
    sum_idx = sum_idx4[n_lin2_groups:].reshape(-1)
    sum_w = sum_w4[n_lin2_groups:].reshape(-1)

    fin = np.isfinite(wtri)
    first = fin.argmax(axis=1)
    width = (fin.sum(axis=1)).astype(np.int64)
    tri_group_start = (n_sum // _LANES) * _LANES
    n_tri_groups = (_N_OUT - tri_group_start) // _LANES
    tidx_chunks, tw_chunks, group_k = [], [], []
    for g in range(n_tri_groups):
        outs = np.arange(tri_group_start + g * _LANES, tri_group_start + (g + 1) * _LANES)
        tri_i = outs - n_sum
        real = tri_i >= 0
        k_g = int(width[tri_i[real]].max())
        gi = np.zeros((k_g, _LANES), np.int32)
        gw = np.full((k_g, _LANES), -np.inf, np.float32)
        for lane in range(_LANES):
            if not real[lane]:
                continue
            i = tri_i[lane]
            f0, wd = int(first[i]), int(width[i])
            gi[:wd, lane] = np.arange(f0, f0 + wd, dtype=np.int32)
            gw[:wd, lane] = wtri[i, f0 : f0 + wd]
        tidx_chunks.append(gi.reshape(-1))
        tw_chunks.append(gw.reshape(-1))
        group_k.append(k_g)
    tri_idx = np.concatenate(tidx_chunks)
    tri_w = np.concatenate(tw_chunks)
    return (
        n_lin2_groups,
        n_sum_groups,
        n_sum - (n_sum // _LANES) * _LANES,
        tuple(group_k),
        lin2_idx,
        lin2_w,
        sum_idx,
        sum_w,
        tri_idx,
        tri_w,
    )


(
    _N_LIN2_GROUPS,
    _N_SUM_GROUPS,
    _N_BOUNDARY,
    _TRI_K,
    _LIN2_IDX,
    _LIN2_W,
    _SUM_IDX,
    _SUM_W,
    _TRI_IDX,
    _TRI_W,
) = _build_tables()

_NUM_CORES = 2
_NUM_SUBCORES = 16
_NW = _NUM_CORES * _NUM_SUBCORES
_ROWS = 8 * 256
_RPW = _ROWS // _NW
_BLK = 16
_NBLK = _RPW // _BLK
_RB = 8


def _sc_body(
    x_hbm, l2i_hbm, l2w_hbm, sidx_hbm, sw_hbm, tidx_hbm, tw_hbm, out_hbm,
    xb, ob, l2i, l2w, sidx, sw, tidx, tw, sems,
):
    cid = lax.axis_index("c")
    sid = lax.axis_index("s")
    wid = sid * _NUM_CORES + cid
    pltpu.sync_copy(l2i_hbm, l2i)
    pltpu.sync_copy(l2w_hbm, l2w)
    pltpu.sync_copy(sidx_hbm, sidx)
    pltpu.sync_copy(sw_hbm, sw)
    pltpu.sync_copy(tidx_hbm, tidx)
    pltpu.sync_copy(tw_hbm, tw)

    def do_subblock(rows, rowoff):
        rbase = [jnp.full((_LANES,), r, jnp.int32) + rowoff for r in rows]

        def lin_group(g, _):
            accs = [jnp.zeros((_LANES,), jnp.float32) for _ in rows]
            for k in range(2):
                off = (g * 2 + k) * _LANES
                iv = l2i[pl.ds(off, _LANES)]
                wv = l2w[pl.ds(off, _LANES)]
                for j, rb in enumerate(rbase):
                    accs[j] = accs[j] + wv * plsc.load_gather(xb, [rb, iv])
            goff = g * _LANES
            for j, r in enumerate(rows):
                ob[r, pl.ds(goff, _LANES)] = accs[j]
            return _

        lax.fori_loop(0, _N_LIN2_GROUPS, lin_group, None)

        n4 = _N_SUM_GROUPS - _N_LIN2_GROUPS

        def sum_group(g, _):
            accs = [jnp.zeros((_LANES,), jnp.float32) for _ in rows]
            for k in range(4):
                off = (g * 4 + k) * _LANES
                iv = sidx[pl.ds(off, _LANES)]
                wv = sw[pl.ds(off, _LANES)]
                for j, rb in enumerate(rbase):
                    accs[j] = accs[j] + wv * plsc.load_gather(xb, [rb, iv])
            goff = (_N_LIN2_GROUPS + g) * _LANES
            for j, r in enumerate(rows):
                ob[r, pl.ds(goff, _LANES)] = accs[j]
            return _

        lax.fori_loop(0, n4 - 1, sum_group, None)

        baccs = [jnp.zeros((_LANES,), jnp.float32) for _ in rows]
        gb4 = n4 - 1
        for k in range(4):
            off = (gb4 * 4 + k) * _LANES
            iv = sidx[pl.ds(off, _LANES)]
            wv = sw[pl.ds(off, _LANES)]
            for j, rb in enumerate(rbase):
                baccs[j] = baccs[j] + wv * plsc.load_gather(xb, [rb, iv])

        gb = _N_SUM_GROUPS - 1
        toff = 0
        for tg, k_g in enumerate(_TRI_K):
            base = toff

            def tri_tap(k, accs, base=base):
                off = (base + k) * _LANES
                iv = tidx[pl.ds(off, _LANES)]
                wv = tw[pl.ds(off, _LANES)]
                return tuple(
                    jnp.maximum(a, wv + plsc.load_gather(xb, [rb, iv]))
                    for a, rb in zip(accs, rbase)
                )

            init = tuple(jnp.full((_LANES,), -jnp.inf, jnp.float32) for _ in rows)
            accs = lax.fori_loop(0, k_g, tri_tap, init)
            toff += k_g
            if tg == 0:
                lane = lax.iota(jnp.int32, _LANES)
                accs = tuple(
                    jnp.where(lane < _N_BOUNDARY, b, a) for a, b in zip(accs, baccs)
                )
            for j, r in enumerate(rows):
                ob[r, pl.ds((gb + tg) * _LANES, _LANES)] = accs[j]

    batch = wid // (256 // _RPW)
    frame0 = (wid % (256 // _RPW)) * _RPW

    def in_copy(blk, slot):
        return pltpu.make_async_copy(
            x_hbm.at[batch, pl.ds(frame0 + blk * _BLK, _BLK)],
            xb.at[pl.ds(slot * _BLK, _BLK)],
            sems.at[slot],
        )

    in_copy(0, 0).start()

    def do_block(blk, _):
        slot = lax.rem(blk, 2)
        in_copy(blk, slot).wait()

        @pl.when(blk < _NBLK - 1)
        def _prefetch():
            in_copy(blk + 1, 1 - slot).start()

        rowoff = slot * _BLK
        for sub in range(_BLK // _RB):
            do_subblock(list(range(sub * _RB, (sub + 1) * _RB)), rowoff)
        pltpu.sync_copy(ob, out_hbm.at[batch, pl.ds(frame0 + blk * _BLK, _BLK)])
        return _

    lax.fori_loop(0, _NBLK, do_block, None)


@jax.jit
def _log_scale_sc(x3d):
    n_tri_taps = _TRI_IDX.shape[0] // _LANES
    call = pl.kernel(
        _sc_body,
        out_type=jax.ShapeDtypeStruct((8, 256, _N_OUT), jnp.float32),
        mesh=plsc.VectorSubcoreMesh(
            core_axis_name="c",
            subcore_axis_name="s",
            num_cores=_NUM_CORES,
            num_subcores=_NUM_SUBCORES,
        ),
        compiler_params=pltpu.CompilerParams(
            use_tc_tiling_on_sc=True,
            needs_layout_passes=False,
            disable_bounds_checks=True,
        ),
        scratch_types=[
            pltpu.VMEM((2 * _BLK, _N_IN), jnp.float32),
            pltpu.VMEM((_BLK, _N_OUT), jnp.float32),
            pltpu.VMEM((_LIN2_IDX.shape[0],), jnp.int32),
            pltpu.VMEM((_LIN2_W.shape[0],), jnp.float32),
            pltpu.VMEM((_SUM_IDX.shape[0],), jnp.int32),
            pltpu.VMEM((_SUM_W.shape[0],), jnp.float32),
            pltpu.VMEM((n_tri_taps * _LANES,), jnp.int32),
            pltpu.VMEM((n_tri_taps * _LANES,), jnp.float32),
            pltpu.SemaphoreType.DMA((2,)),
        ],
    )
    return call(x3d, _LIN2_IDX, _LIN2_W, _SUM_IDX, _SUM_W, _TRI_IDX, _TRI_W)


def kernel(x):
    return _log_scale_sc(x)


# speedup vs baseline: 1.5712x; 1.0270x over previous
import functools
import math

import numpy as np
import jax
import jax.numpy as jnp
from jax import lax
from jax.experimental import pallas as pl
from jax.experimental.pallas import tpu as pltpu
from jax.experimental.pallas import tpu_sc as plsc

_N_IN = 2049
_N_OUT = 512
_OUTPUT_START = 20.0
_OUTPUT_END = 20000.0
_INPUT_END = 24000.0
_LANES = 16


def _op_constants():
    min_log = math.log10(1.0 + _OUTPUT_START)
    max_log = math.log10(1.0 + _OUTPUT_END)
    lin_logs = np.linspace(min_log, max_log, _N_OUT, dtype=np.float64)
    freq_per_bin = float(_INPUT_END) / (_N_IN - 1)
    center = ((np.power(10.0, lin_logs) - 1.0) / freq_per_bin).astype(np.float32)
    n_linear = 0
    while n_linear < _N_OUT - 1 and (
        center[n_linear + 1] - center[n_linear] <= 1.0 or center[n_linear] < 1.0
    ):
        n_linear += 1
    start_idx = center[:n_linear].astype(np.int64)
    frac_linear = center[:n_linear] - start_idx.astype(np.float32)
    n_sum = n_linear
    while n_sum < _N_OUT - 2 and (
        center[n_sum + 1] - center[n_sum] <= 2.0 or center[n_sum] < 2.0
    ):
        n_sum += 1
    n_cubic = n_sum - n_linear
    frac_cubic = center[n_linear : n_linear + n_cubic].copy()
    n_tri = _N_OUT - n_sum
    frac_tri = center[n_sum : n_sum + n_tri].copy()
    dist_tri = (center[n_sum : n_sum + n_tri] - center[n_sum - 1 : n_sum - 1 + n_tri]).copy()
    weights = np.full((n_tri, _N_IN), -np.inf, dtype=np.float32)
    for i in range(n_tri):
        i_mid = int(math.ceil(frac_tri[i]))
        i_start = int(math.ceil(frac_tri[i] - dist_tri[i]))
        i_end = int(math.ceil(frac_tri[i] + dist_tri[i + 1])) if i < n_tri - 1 else i_mid
        weights[i, i_mid] = 0.0
        dist_left = float(i_mid - i_start)
        for i_bin in range(i_start + 1, i_mid):
            lw = 1.0 - (i_mid - i_bin) / dist_left
            weights[i, i_bin] = np.float32(10.0 * np.log10(np.float32(lw)))
        if i_end > i_mid:
            dist_right = float(i_end - i_mid)
            for i_bin in range(i_mid + 1, i_end):
                lw = np.float32(1.0 - (i_bin - i_mid) / dist_right)
                weights[i, i_bin] = np.float32(10.0 * np.log10(lw))
    return n_linear, n_cubic, n_tri, start_idx, frac_linear, frac_cubic, weights


def _build_tables():
    n_linear, n_cubic, n_tri, lin_i0, frac_lin, frac_cub, wtri = _op_constants()
    n_sum = n_linear + n_cubic
    n_sum_groups = (n_sum + _LANES - 1) // _LANES
    sum_idx = np.zeros((n_sum_groups * _LANES, 4), np.int32)
    sum_w = np.zeros((n_sum_groups * _LANES, 4), np.float32)
    sum_idx[:n_linear, 0] = lin_i0
    sum_w[:n_linear, 0] = 1.0 - frac_lin
    sum_idx[:n_linear, 1] = lin_i0 + 1
    sum_w[:n_linear, 1] = frac_lin
    ci = np.floor(frac_cub).astype(np.int64)
    t = (frac_cub - ci.astype(np.float32)).astype(np.float32)
    cw = np.stack(
        [
            0.5 * (-t + 2 * t**2 - t**3),
            0.5 * (2 - 5 * t**2 + 3 * t**3),
            0.5 * (t + 4 * t**2 - 3 * t**3),
            0.5 * (-(t**2) + t**3),
        ],
        axis=1,
    ).astype(np.float32)
    for k in range(4):
        sum_idx[n_linear:n_sum, k] = np.clip(ci - 1 + k, 0, _N_IN - 1)
        sum_w[n_linear:n_sum, k] = cw[:, k]
    n_lin2_groups = n_linear // _LANES
    sum_idx4 = sum_idx.reshape(n_sum_groups, _LANES, 4).transpose(0, 2, 1)
    sum_w4 = sum_w.reshape(n_sum_groups, _LANES, 4).transpose(0, 2, 1)
    lin2_idx = sum_idx4[:n_lin2_groups, :2].reshape(-1)
    lin2_w = sum_w4[:n_lin2_groups, :2].reshape(-1)
    sum_idx = sum_idx4[n_lin2_groups:].reshape(-1)
    sum_w = sum_w4[n_lin2_groups:].reshape(-1)

    fin = np.isfinite(wtri)
    first = fin.argmax(axis=1)
    width = (fin.sum(axis=1)).astype(np.int64)
    tri_group_start = (n_sum // _LANES) * _LANES
    n_tri_groups = (_N_OUT - tri_group_start) // _LANES
    tidx_chunks, tw_chunks, group_k = [], [], []
    for g in range(n_tri_groups):
        outs = np.arange(tri_group_start + g * _LANES, tri_group_start + (g + 1) * _LANES)
        tri_i = outs - n_sum
        real = tri_i >= 0
        k_g = int(width[tri_i[real]].max())
        gi = np.zeros((k_g, _LANES), np.int32)
        gw = np.full((k_g, _LANES), -np.inf, np.float32)
        for lane in range(_LANES):
            if not real[lane]:
                continue
            i = tri_i[lane]
            f0, wd = int(first[i]), int(width[i])
            gi[:wd, lane] = np.arange(f0, f0 + wd, dtype=np.int32)
            gw[:wd, lane] = wtri[i, f0 : f0 + wd]
        tidx_chunks.append(gi.reshape(-1))
        tw_chunks.append(gw.reshape(-1))
        group_k.append(k_g)
    tri_idx = np.concatenate(tidx_chunks)
    tri_w = np.concatenate(tw_chunks)
    return (
        n_lin2_groups,
        n_sum_groups,
        n_sum - (n_sum // _LANES) * _LANES,
        tuple(group_k),
        lin2_idx,
        lin2_w,
        sum_idx,
        sum_w,
        tri_idx,
        tri_w,
    )


(
    _N_LIN2_GROUPS,
    _N_SUM_GROUPS,
    _N_BOUNDARY,
    _TRI_K,
    _LIN2_IDX,
    _LIN2_W,
    _SUM_IDX,
    _SUM_W,
    _TRI_IDX,
    _TRI_W,
) = _build_tables()

_NUM_CORES = 2
_NUM_SUBCORES = 16
_NW = _NUM_CORES * _NUM_SUBCORES
_ROWS = 8 * 256
_RPW = _ROWS // _NW
_BLK = 16
_NBLK = _RPW // _BLK
_RB = 16


def _sc_body(
    x_hbm, l2i_hbm, l2w_hbm, sidx_hbm, sw_hbm, tidx_hbm, tw_hbm, out_hbm,
    xb, ob, l2i, l2w, sidx, sw, tidx, tw, sems,
):
    cid = lax.axis_index("c")
    sid = lax.axis_index("s")
    wid = sid * _NUM_CORES + cid
    pltpu.sync_copy(l2i_hbm, l2i)
    pltpu.sync_copy(l2w_hbm, l2w)
    pltpu.sync_copy(sidx_hbm, sidx)
    pltpu.sync_copy(sw_hbm, sw)
    pltpu.sync_copy(tidx_hbm, tidx)
    pltpu.sync_copy(tw_hbm, tw)

    def do_subblock(rows, rowoff):
        rbase = [jnp.full((_LANES,), r, jnp.int32) + rowoff for r in rows]

        def lin_group(g, _):
            accs = [jnp.zeros((_LANES,), jnp.float32) for _ in rows]
            for k in range(2):
                off = (g * 2 + k) * _LANES
                iv = l2i[pl.ds(off, _LANES)]
                wv = l2w[pl.ds(off, _LANES)]
                for j, rb in enumerate(rbase):
                    accs[j] = accs[j] + wv * plsc.load_gather(xb, [rb, iv])
            goff = g * _LANES
            for j, r in enumerate(rows):
                ob[r, pl.ds(goff, _LANES)] = accs[j]
            return _

        lax.fori_loop(0, _N_LIN2_GROUPS, lin_group, None)

        n4 = _N_SUM_GROUPS - _N_LIN2_GROUPS

        def sum_group(g, _):
            accs = [jnp.zeros((_LANES,), jnp.float32) for _ in rows]
            for k in range(4):
                off = (g * 4 + k) * _LANES
                iv = sidx[pl.ds(off, _LANES)]
                wv = sw[pl.ds(off, _LANES)]
                for j, rb in enumerate(rbase):
                    accs[j] = accs[j] + wv * plsc.load_gather(xb, [rb, iv])
            goff = (_N_LIN2_GROUPS + g) * _LANES
            for j, r in enumerate(rows):
                ob[r, pl.ds(goff, _LANES)] = accs[j]
            return _

        lax.fori_loop(0, n4 - 1, sum_group, None)

        baccs = [jnp.zeros((_LANES,), jnp.float32) for _ in rows]
        gb4 = n4 - 1
        for k in range(4):
            off = (gb4 * 4 + k) * _LANES
            iv = sidx[pl.ds(off, _LANES)]
            wv = sw[pl.ds(off, _LANES)]
            for j, rb in enumerate(rbase):
                baccs[j] = baccs[j] + wv * plsc.load_gather(xb, [rb, iv])

        gb = _N_SUM_GROUPS - 1
        toff = 0
        for tg, k_g in enumerate(_TRI_K):
            base = toff

            def tri_tap(k, accs, base=base):
                off = (base + k) * _LANES
                iv = tidx[pl.ds(off, _LANES)]
                wv = tw[pl.ds(off, _LANES)]
                return tuple(
                    jnp.maximum(a, wv + plsc.load_gather(xb, [rb, iv]))
                    for a, rb in zip(accs, rbase)
                )

            init = tuple(jnp.full((_LANES,), -jnp.inf, jnp.float32) for _ in rows)
            accs = lax.fori_loop(0, k_g, tri_tap, init)
            toff += k_g
            if tg == 0:
                lane = lax.iota(jnp.int32, _LANES)
                accs = tuple(
                    jnp.where(lane < _N_BOUNDARY, b, a) for a, b in zip(accs, baccs)
                )
            for j, r in enumerate(rows):
                ob[r, pl.ds((gb + tg) * _LANES, _LANES)] = accs[j]

    batch = wid // (256 // _RPW)
    frame0 = (wid % (256 // _RPW)) * _RPW

    def in_copy(blk, slot):
        return pltpu.make_async_copy(
            x_hbm.at[batch, pl.ds(frame0 + blk * _BLK, _BLK)],
            xb.at[pl.ds(slot * _BLK, _BLK)],
            sems.at[slot],
        )

    in_copy(0, 0).start()

    def do_block(blk, _):
        slot = lax.rem(blk, 2)
        in_copy(blk, slot).wait()

        @pl.when(blk < _NBLK - 1)
        def _prefetch():
            in_copy(blk + 1, 1 - slot).start()

        rowoff = slot * _BLK
        for sub in range(_BLK // _RB):
            do_subblock(list(range(sub * _RB, (sub + 1) * _RB)), rowoff)
        pltpu.sync_copy(ob, out_hbm.at[batch, pl.ds(frame0 + blk * _BLK, _BLK)])
        return _

    lax.fori_loop(0, _NBLK, do_block, None)


@jax.jit
def _log_scale_sc(x3d):
    n_tri_taps = _TRI_IDX.shape[0] // _LANES
    call = pl.kernel(
        _sc_body,
        out_type=jax.ShapeDtypeStruct((8, 256, _N_OUT), jnp.float32),
        mesh=plsc.VectorSubcoreMesh(
            core_axis_name="c",
            subcore_axis_name="s",
            num_cores=_NUM_CORES,
            num_subcores=_NUM_SUBCORES,
        ),
        compiler_params=pltpu.CompilerParams(
            use_tc_tiling_on_sc=True,
            needs_layout_passes=False,
            disable_bounds_checks=True,
        ),
        scratch_types=[
            pltpu.VMEM((2 * _BLK, _N_IN), jnp.float32),
            pltpu.VMEM((_BLK, _N_OUT), jnp.float32),
            pltpu.VMEM((_LIN2_IDX.shape[0],), jnp.int32),
            pltpu.VMEM((_LIN2_W.shape[0],), jnp.float32),
            pltpu.VMEM((_SUM_IDX.shape[0],), jnp.int32),
            pltpu.VMEM((_SUM_W.shape[0],), jnp.float32),
            pltpu.VMEM((n_tri_taps * _LANES,), jnp.int32),
            pltpu.VMEM((n_tri_taps * _LANES,), jnp.float32),
            pltpu.SemaphoreType.DMA((2,)),
        ],
    )
    return call(x3d, _LIN2_IDX, _LIN2_W, _SUM_IDX, _SUM_W, _TRI_IDX, _TRI_W)


def kernel(x):
    return _log_scale_sc(x)


# speedup vs baseline: 1.6228x; 1.0328x over previous
import functools
import math

import numpy as np
import jax
import jax.numpy as jnp
from jax import lax
from jax.experimental import pallas as pl
from jax.experimental.pallas import tpu as pltpu
from jax.experimental.pallas import tpu_sc as plsc

_N_IN = 2049
_N_OUT = 512
_OUTPUT_START = 20.0
_OUTPUT_END = 20000.0
_INPUT_END = 24000.0
_LANES = 16


def _op_constants():
    min_log = math.log10(1.0 + _OUTPUT_START)
    max_log = math.log10(1.0 + _OUTPUT_END)
    lin_logs = np.linspace(min_log, max_log, _N_OUT, dtype=np.float64)
    freq_per_bin = float(_INPUT_END) / (_N_IN - 1)
    center = ((np.power(10.0, lin_logs) - 1.0) / freq_per_bin).astype(np.float32)
    n_linear = 0
    while n_linear < _N_OUT - 1 and (
        center[n_linear + 1] - center[n_linear] <= 1.0 or center[n_linear] < 1.0
    ):
        n_linear += 1
    start_idx = center[:n_linear].astype(np.int64)
    frac_linear = center[:n_linear] - start_idx.astype(np.float32)
    n_sum = n_linear
    while n_sum < _N_OUT - 2 and (
        center[n_sum + 1] - center[n_sum] <= 2.0 or center[n_sum] < 2.0
    ):
        n_sum += 1
    n_cubic = n_sum - n_linear
    frac_cubic = center[n_linear : n_linear + n_cubic].copy()
    n_tri = _N_OUT - n_sum
    frac_tri = center[n_sum : n_sum + n_tri].copy()
    dist_tri = (center[n_sum : n_sum + n_tri] - center[n_sum - 1 : n_sum - 1 + n_tri]).copy()
    weights = np.full((n_tri, _N_IN), -np.inf, dtype=np.float32)
    for i in range(n_tri):
        i_mid = int(math.ceil(frac_tri[i]))
        i_start = int(math.ceil(frac_tri[i] - dist_tri[i]))
        i_end = int(math.ceil(frac_tri[i] + dist_tri[i + 1])) if i < n_tri - 1 else i_mid
        weights[i, i_mid] = 0.0
        dist_left = float(i_mid - i_start)
        for i_bin in range(i_start + 1, i_mid):
            lw = 1.0 - (i_mid - i_bin) / dist_left
            weights[i, i_bin] = np.float32(10.0 * np.log10(np.float32(lw)))
        if i_end > i_mid:
            dist_right = float(i_end - i_mid)
            for i_bin in range(i_mid + 1, i_end):
                lw = np.float32(1.0 - (i_bin - i_mid) / dist_right)
                weights[i, i_bin] = np.float32(10.0 * np.log10(lw))
    return n_linear, n_cubic, n_tri, start_idx, frac_linear, frac_cubic, weights


def _build_tables():
    n_linear, n_cubic, n_tri, lin_i0, frac_lin, frac_cub, wtri = _op_constants()
    n_sum = n_linear + n_cubic
    n_sum_groups = (n_sum + _LANES - 1) // _LANES
    sum_idx = np.zeros((n_sum_groups * _LANES, 4), np.int32)
    sum_w = np.zeros((n_sum_groups * _LANES, 4), np.float32)
    sum_idx[:n_linear, 0] = lin_i0
    sum_w[:n_linear, 0] = 1.0 - frac_lin
    sum_idx[:n_linear, 1] = lin_i0 + 1
    sum_w[:n_linear, 1] = frac_lin
    ci = np.floor(frac_cub).astype(np.int64)
    t = (frac_cub - ci.astype(np.float32)).astype(np.float32)
    cw = np.stack(
        [
            0.5 * (-t + 2 * t**2 - t**3),
            0.5 * (2 - 5 * t**2 + 3 * t**3),
            0.5 * (t + 4 * t**2 - 3 * t**3),
            0.5 * (-(t**2) + t**3),
        ],
        axis=1,
    ).astype(np.float32)
    for k in range(4):
        sum_idx[n_linear:n_sum, k] = np.clip(ci - 1 + k, 0, _N_IN - 1)
        sum_w[n_linear:n_sum, k] = cw[:, k]
    n_lin2_groups = n_linear // _LANES
    sum_idx4 = sum_idx.reshape(n_sum_groups, _LANES, 4).transpose(0, 2, 1)
    sum_w4 = sum_w.reshape(n_sum_groups, _LANES, 4).transpose(0, 2, 1)
    lin2_idx = sum_idx4[:n_lin2_groups, :2].reshape(-1)
    lin2_w = sum_w4[:n_lin2_groups, :2].reshape(-1)
    sum_idx = sum_idx4[n_lin2_groups:].reshape(-1)
    sum_w = sum_w4[n_lin2_groups:].reshape(-1)

    fin = np.isfinite(wtri)
    first = fin.argmax(axis=1)
    width = (fin.sum(axis=1)).astype(np.int64)
    tri_group_start = (n_sum // _LANES) * _LANES
    n_tri_groups = (_N_OUT - tri_group_start) // _LANES
    tidx_chunks, tw_chunks, group_k = [], [], []
    for g in range(n_tri_groups):
        outs = np.arange(tri_group_start + g * _LANES, tri_group_start + (g + 1) * _LANES)
        tri_i = outs - n_sum
        real = tri_i >= 0
        k_g = int(width[tri_i[real]].max())
        gi = np.zeros((k_g, _LANES), np.int32)
        gw = np.full((k_g, _LANES), -np.inf, np.float32)
        for lane in range(_LANES):
            if not real[lane]:
                continue
            i = tri_i[lane]
            f0, wd = int(first[i]), int(width[i])
            gi[:wd, lane] = np.arange(f0, f0 + wd, dtype=np.int32)
            gw[:wd, lane] = wtri[i, f0 : f0 + wd]
        tidx_chunks.append(gi.reshape(-1))
        tw_chunks.append(gw.reshape(-1))
        group_k.append(k_g)
    tri_idx = np.concatenate(tidx_chunks)
    tri_w = np.concatenate(tw_chunks)
    return (
        n_lin2_groups,
        n_sum_groups,
        n_sum - (n_sum // _LANES) * _LANES,
        tuple(group_k),
        lin2_idx,
        lin2_w,
        sum_idx,
        sum_w,
        tri_idx,
        tri_w,
    )


(
    _N_LIN2_GROUPS,
    _N_SUM_GROUPS,
    _N_BOUNDARY,
    _TRI_K,
    _LIN2_IDX,
    _LIN2_W,
    _SUM_IDX,
    _SUM_W,
    _TRI_IDX,
    _TRI_W,
) = _build_tables()

_NUM_CORES = 2
_NUM_SUBCORES = 16
_NW = _NUM_CORES * _NUM_SUBCORES
_ROWS = 8 * 256
_RPW = _ROWS // _NW
_BLK = 16
_NBLK = _RPW // _BLK
_RB = 16


def _sc_body(
    x_hbm, l2i_hbm, l2w_hbm, sidx_hbm, sw_hbm, tidx_hbm, tw_hbm, out_hbm,
    xb, ob, l2i, l2w, sidx, sw, tidx, tw, sems,
):
    cid = lax.axis_index("c")
    sid = lax.axis_index("s")
    wid = sid * _NUM_CORES + cid
    tab_copies = [
        pltpu.make_async_copy(src, dst, sems.at[0])
        for src, dst in (
            (l2i_hbm, l2i),
            (l2w_hbm, l2w),
            (sidx_hbm, sidx),
            (sw_hbm, sw),
            (tidx_hbm, tidx),
            (tw_hbm, tw),
        )
    ]
    for c in tab_copies:
        c.start()
    for c in tab_copies:
        c.wait()

    def do_subblock(rows, rowoff):
        rbase = [jnp.full((_LANES,), r, jnp.int32) + rowoff for r in rows]

        def lin_group(g, _):
            accs = [jnp.zeros((_LANES,), jnp.float32) for _ in rows]
            for k in range(2):
                off = (g * 2 + k) * _LANES
                iv = l2i[pl.ds(off, _LANES)]
                wv = l2w[pl.ds(off, _LANES)]
                for j, rb in enumerate(rbase):
                    accs[j] = accs[j] + wv * plsc.load_gather(xb, [rb, iv])
            goff = g * _LANES
            for j, r in enumerate(rows):
                ob[r, pl.ds(goff, _LANES)] = accs[j]
            return _

        lax.fori_loop(0, _N_LIN2_GROUPS, lin_group, None)

        n4 = _N_SUM_GROUPS - _N_LIN2_GROUPS

        def sum_group(g, _):
            accs = [jnp.zeros((_LANES,), jnp.float32) for _ in rows]
            for k in range(4):
                off = (g * 4 + k) * _LANES
                iv = sidx[pl.ds(off, _LANES)]
                wv = sw[pl.ds(off, _LANES)]
                for j, rb in enumerate(rbase):
                    accs[j] = accs[j] + wv * plsc.load_gather(xb, [rb, iv])
            goff = (_N_LIN2_GROUPS + g) * _LANES
            for j, r in enumerate(rows):
                ob[r, pl.ds(goff, _LANES)] = accs[j]
            return _

        lax.fori_loop(0, n4 - 1, sum_group, None)

        baccs = [jnp.zeros((_LANES,), jnp.float32) for _ in rows]
        gb4 = n4 - 1
        for k in range(4):
            off = (gb4 * 4 + k) * _LANES
            iv = sidx[pl.ds(off, _LANES)]
            wv = sw[pl.ds(off, _LANES)]
            for j, rb in enumerate(rbase):
                baccs[j] = baccs[j] + wv * plsc.load_gather(xb, [rb, iv])

        gb = _N_SUM_GROUPS - 1
        toff = 0
        for tg, k_g in enumerate(_TRI_K):
            base = toff

            def tri_tap(k, accs, base=base):
                off = (base + k) * _LANES
                iv = tidx[pl.ds(off, _LANES)]
                wv = tw[pl.ds(off, _LANES)]
                return tuple(
                    jnp.maximum(a, wv + plsc.load_gather(xb, [rb, iv]))
                    for a, rb in zip(accs, rbase)
                )

            init = tuple(jnp.full((_LANES,), -jnp.inf, jnp.float32) for _ in rows)
            accs = lax.fori_loop(0, k_g, tri_tap, init)
            toff += k_g
            if tg == 0:
                lane = lax.iota(jnp.int32, _LANES)
                accs = tuple(
                    jnp.where(lane < _N_BOUNDARY, b, a) for a, b in zip(accs, baccs)
                )
            for j, r in enumerate(rows):
                ob[r, pl.ds((gb + tg) * _LANES, _LANES)] = accs[j]

    batch = wid // (256 // _RPW)
    frame0 = (wid % (256 // _RPW)) * _RPW

    def in_copy(blk, slot):
        return pltpu.make_async_copy(
            x_hbm.at[batch, pl.ds(frame0 + blk * _BLK, _BLK)],
            xb.at[pl.ds(slot * _BLK, _BLK)],
            sems.at[slot],
        )

    in_copy(0, 0).start()

    def do_block(blk, _):
        slot = lax.rem(blk, 2)
        in_copy(blk, slot).wait()

        @pl.when(blk < _NBLK - 1)
        def _prefetch():
            in_copy(blk + 1, 1 - slot).start()

        rowoff = slot * _BLK
        for sub in range(_BLK // _RB):
            do_subblock(list(range(sub * _RB, (sub + 1) * _RB)), rowoff)
        pltpu.sync_copy(ob, out_hbm.at[batch, pl.ds(frame0 + blk * _BLK, _BLK)])
        return _

    lax.fori_loop(0, _NBLK, do_block, None)


@jax.jit
def _log_scale_sc(x3d):
    n_tri_taps = _TRI_IDX.shape[0] // _LANES
    call = pl.kernel(
        _sc_body,
        out_type=jax.ShapeDtypeStruct((8, 256, _N_OUT), jnp.float32),
        mesh=plsc.VectorSubcoreMesh(
            core_axis_name="c",
            subcore_axis_name="s",
            num_cores=_NUM_CORES,
            num_subcores=_NUM_SUBCORES,
        ),
        compiler_params=pltpu.CompilerParams(
            use_tc_tiling_on_sc=True,
            needs_layout_passes=False,
            disable_bounds_checks=True,
        ),
        scratch_types=[
            pltpu.VMEM((2 * _BLK, _N_IN), jnp.float32),
            pltpu.VMEM((_BLK, _N_OUT), jnp.float32),
            pltpu.VMEM((_LIN2_IDX.shape[0],), jnp.int32),
            pltpu.VMEM((_LIN2_W.shape[0],), jnp.float32),
            pltpu.VMEM((_SUM_IDX.shape[0],), jnp.int32),
            pltpu.VMEM((_SUM_W.shape[0],), jnp.float32),
            pltpu.VMEM((n_tri_taps * _LANES,), jnp.int32),
            pltpu.VMEM((n_tri_taps * _LANES,), jnp.float32),
            pltpu.SemaphoreType.DMA((2,)),
        ],
    )
    return call(x3d, _LIN2_IDX, _LIN2_W, _SUM_IDX, _SUM_W, _TRI_IDX, _TRI_W)


def kernel(x):
    return _log_scale_sc(x)
